# baseline (speedup 1.0000x reference)
import functools

import jax
import jax.numpy as jnp
from jax import lax
from jax.experimental import pallas as pl
from jax.experimental.pallas import tpu as pltpu
from jax.experimental.pallas import tpu_sc as plsc

N = 320000
D = 128
S = 1024

NC = 2
NS = 16
NW = NC * NS
ROWS_PER_W = N // NW
CHUNK = 125
NCHUNK = ROWS_PER_W // CHUNK
NBUF = 4
TAIL = NBUF * ((NCHUNK - 1) // NBUF)
ROWS_PER_TILE_OUT = S // NS
ZROWS = 16


def _sc_body(x_hbm, batch_hbm, out_hbm, xb0, xb1, xb2, xb3, ibuf, zbuf, acc,
             sx0, sx1, sx2, sx3, ss0, ss1, ss2, ss3):
    xb = (xb0, xb1, xb2, xb3)
    sx = (sx0, sx1, sx2, sx3)
    ss = (ss0, ss1, ss2, ss3)

    c = lax.axis_index("c")
    s = lax.axis_index("s")
    wid = c * NS + s

    pltpu.async_copy(batch_hbm.at[wid], ibuf, sx[NBUF - 1])

    def zrow(i, _):
        for j in range(D // 16):
            zbuf[i, pl.ds(j * 16, 16)] = jnp.zeros((16,), jnp.float32)
        return 0
    lax.fori_loop(0, ZROWS, zrow, 0)
    for k in range(ROWS_PER_TILE_OUT // ZROWS):
        pltpu.sync_copy(
            zbuf, acc.at[pl.ds(s * ROWS_PER_TILE_OUT + k * ZROWS, ZROWS)])
    pltpu.make_async_copy(batch_hbm.at[wid], ibuf, sx[NBUF - 1]).wait()
    plsc.subcore_barrier()

    def start_load(ch, b):
        pltpu.async_copy(x_hbm.at[wid, ch], xb[b], sx[b])

    def wait_load(ch, b):
        pltpu.make_async_copy(x_hbm.at[wid, ch], xb[b], sx[b]).wait()

    def start_scatter(ch, b):
        pltpu.async_copy(xb[b], acc.at[ibuf.at[ch]], ss[b], add=True)

    def wait_scatter(ch, b):
        pltpu.make_async_copy(xb[b], acc.at[ibuf.at[ch]], ss[b]).wait()

    for p in range(NBUF - 1):
        start_load(p, p)

    def outer(k, _):
        for b in range(NBUF):
            ch = NBUF * k + b
            wait_load(ch, b)
            start_scatter(ch, b)

            @pl.when(ch >= 1)
            def _():
                wait_scatter(ch - 1, (b - 1) % NBUF)

            @pl.when(ch + NBUF - 1 < NCHUNK)
            def _():
                start_load(ch + NBUF - 1, (b - 1) % NBUF)
        return 0

    lax.fori_loop(0, (NCHUNK - 1) // NBUF, outer, 0)

    for ch in range(TAIL, NCHUNK):
        b = ch % NBUF
        wait_load(ch, b)
        start_scatter(ch, b)
        if ch >= 1:
            wait_scatter(ch - 1, (ch - 1) % NBUF)
        if ch + NBUF - 1 < NCHUNK:
            start_load(ch + NBUF - 1, (b - 1) % NBUF)
    wait_scatter(NCHUNK - 1, (NCHUNK - 1) % NBUF)

    plsc.subcore_barrier()
    pltpu.sync_copy(
        acc.at[pl.ds(s * ROWS_PER_TILE_OUT, ROWS_PER_TILE_OUT)],
        out_hbm.at[c, pl.ds(s * ROWS_PER_TILE_OUT, ROWS_PER_TILE_OUT)],
    )


_sc_stage = functools.partial(
    pl.kernel,
    out_type=jax.ShapeDtypeStruct((NC, S, D), jnp.float32),
    mesh=plsc.VectorSubcoreMesh(core_axis_name="c", subcore_axis_name="s"),
    scratch_types=[
        pltpu.VMEM((CHUNK, D), jnp.float32),
        pltpu.VMEM((CHUNK, D), jnp.float32),
        pltpu.VMEM((CHUNK, D), jnp.float32),
        pltpu.VMEM((CHUNK, D), jnp.float32),
        pltpu.VMEM((NCHUNK, CHUNK), jnp.int32),
        pltpu.VMEM((ZROWS, D), jnp.float32),
        pltpu.VMEM_SHARED((S, D), jnp.float32),
        pltpu.SemaphoreType.DMA,
        pltpu.SemaphoreType.DMA,
        pltpu.SemaphoreType.DMA,
        pltpu.SemaphoreType.DMA,
        pltpu.SemaphoreType.DMA,
        pltpu.SemaphoreType.DMA,
        pltpu.SemaphoreType.DMA,
        pltpu.SemaphoreType.DMA,
    ],
)(_sc_body)


def _add_body(p_ref, o_ref):
    o_ref[...] = p_ref[0] + p_ref[1]


def kernel(x, batch):
    xr = x.reshape(NW, NCHUNK, CHUNK, D)
    br = batch.reshape(NW, NCHUNK, CHUNK)
    partials = _sc_stage(xr, br)
    out = pl.pallas_call(
        _add_body,
        out_shape=jax.ShapeDtypeStruct((S, D), jnp.float32),
    )(partials)
    return out


# speedup vs baseline: 2.0665x; 2.0665x over previous
import functools

import jax
import jax.numpy as jnp
from jax import lax
from jax.experimental import pallas as pl
from jax.experimental.pallas import tpu as pltpu
from jax.experimental.pallas import tpu_sc as plsc

N = 320000
D = 128
S = 1024

NC = 2
NS = 16
NW = NC * NS
ROWS_PER_W = N // NW
CHUNK = 80
NCHUNK = ROWS_PER_W // CHUNK
NBUF = 6
TAIL = NBUF * ((NCHUNK - 1) // NBUF)
ROWS_PER_TILE_OUT = S // NS
ZROWS = 16


def _sc_body(x_hbm, batch_hbm, out_hbm, xb0, xb1, xb2, xb3, xb4, xb5,
             ibuf, zbuf, acc,
             sx0, sx1, sx2, sx3, sx4, sx5, ss0, ss1, ss2, ss3, ss4, ss5):
    xb = (xb0, xb1, xb2, xb3, xb4, xb5)
    sx = (sx0, sx1, sx2, sx3, sx4, sx5)
    ss = (ss0, ss1, ss2, ss3, ss4, ss5)

    c = lax.axis_index("c")
    s = lax.axis_index("s")
    wid = c * NS + s

    pltpu.async_copy(batch_hbm.at[wid], ibuf, sx[NBUF - 1])

    def zrow(i, _):
        for j in range(D // 16):
            zbuf[i, pl.ds(j * 16, 16)] = jnp.zeros((16,), jnp.float32)
        return 0
    lax.fori_loop(0, ZROWS, zrow, 0)
    for k in range(ROWS_PER_TILE_OUT // ZROWS):
        pltpu.sync_copy(
            zbuf, acc.at[pl.ds(s * ROWS_PER_TILE_OUT + k * ZROWS, ZROWS)])
    pltpu.make_async_copy(batch_hbm.at[wid], ibuf, sx[NBUF - 1]).wait()
    plsc.subcore_barrier()

    def start_load(ch, b):
        pltpu.async_copy(x_hbm.at[wid, ch], xb[b], sx[b])

    def wait_load(ch, b):
        pltpu.make_async_copy(x_hbm.at[wid, ch], xb[b], sx[b]).wait()

    def start_scatter(ch, b):
        pltpu.async_copy(xb[b], acc.at[ibuf.at[ch]], ss[b], add=True)

    def wait_scatter(ch, b):
        pltpu.make_async_copy(xb[b], acc.at[ibuf.at[ch]], ss[b]).wait()

    for p in range(NBUF - 1):
        start_load(p, p)

    def outer(k, _):
        for b in range(NBUF):
            ch = NBUF * k + b
            wait_load(ch, b)
            start_scatter(ch, b)

            @pl.when(ch >= 1)
            def _():
                wait_scatter(ch - 1, (b - 1) % NBUF)

            @pl.when(ch + NBUF - 1 < NCHUNK)
            def _():
                start_load(ch + NBUF - 1, (b - 1) % NBUF)
        return 0

    lax.fori_loop(0, (NCHUNK - 1) // NBUF, outer, 0)

    for ch in range(TAIL, NCHUNK):
        b = ch % NBUF
        wait_load(ch, b)
        start_scatter(ch, b)
        if ch >= 1:
            wait_scatter(ch - 1, (ch - 1) % NBUF)
        if ch + NBUF - 1 < NCHUNK:
            start_load(ch + NBUF - 1, (b - 1) % NBUF)
    wait_scatter(NCHUNK - 1, (NCHUNK - 1) % NBUF)

    plsc.subcore_barrier()
    pltpu.sync_copy(
        acc.at[pl.ds(s * ROWS_PER_TILE_OUT, ROWS_PER_TILE_OUT)],
        out_hbm.at[c, pl.ds(s * ROWS_PER_TILE_OUT, ROWS_PER_TILE_OUT)],
    )


_sc_stage = functools.partial(
    pl.kernel,
    out_type=jax.ShapeDtypeStruct((NC, S, D), jnp.float32),
    mesh=plsc.VectorSubcoreMesh(core_axis_name="c", subcore_axis_name="s"),
    scratch_types=[
        pltpu.VMEM((CHUNK, D), jnp.float32),
        pltpu.VMEM((CHUNK, D), jnp.float32),
        pltpu.VMEM((CHUNK, D), jnp.float32),
        pltpu.VMEM((CHUNK, D), jnp.float32),
        pltpu.VMEM((CHUNK, D), jnp.float32),
        pltpu.VMEM((CHUNK, D), jnp.float32),
        pltpu.VMEM((NCHUNK, CHUNK), jnp.int32),
        pltpu.VMEM((ZROWS, D), jnp.float32),
        pltpu.VMEM_SHARED((S, D), jnp.float32),
    ] + [pltpu.SemaphoreType.DMA] * 12,
)(_sc_body)


def _add_body(p_ref, o_ref):
    o_ref[...] = p_ref[0] + p_ref[1]


def kernel(x, batch):
    xr = x.reshape(NW, NCHUNK, CHUNK, D)
    br = batch.reshape(NW, NCHUNK, CHUNK)
    partials = _sc_stage(xr, br)
    out = pl.pallas_call(
        _add_body,
        out_shape=jax.ShapeDtypeStruct((S, D), jnp.float32),
    )(partials)
    return out


# speedup vs baseline: 2.5487x; 1.2333x over previous
import functools

import jax
import jax.numpy as jnp
from jax import lax
from jax.experimental import pallas as pl
from jax.experimental.pallas import tpu as pltpu
from jax.experimental.pallas import tpu_sc as plsc

N = 320000
D = 128
NL = D // 16
S = 1024

NC = 2
NS = 16
NW = NC * NS
ROWS_PER_W = N // NW
CHUNK = 80
NCHUNK = ROWS_PER_W // CHUNK
NBUF = 5
NPART = 4
ROWS_PER_TILE_OUT = S // NS
ZROWS = 16


def _sc_body(x_hbm, batch_hbm, out_hbm,
             xb0, xb1, xb2, xb3, xb4, xs0, xs1, xs2, xs3, xs4,
             ibuf, ib4, zbuf, acc,
             sx0, sx1, sx2, sx3, sx4, ss0, ss1, ss2, ss3, ss4):
    xb = (xb0, xb1, xb2, xb3, xb4)
    xs = (xs0, xs1, xs2, xs3, xs4)
    sx = (sx0, sx1, sx2, sx3, sx4)
    ss = (ss0, ss1, ss2, ss3, ss4)

    c = lax.axis_index("c")
    s = lax.axis_index("s")
    wid = c * NS + s
    zero16 = jnp.zeros((16,), jnp.float32)

    pltpu.async_copy(batch_hbm.at[wid], ibuf, sx[NBUF - 1])

    def zrow(i, _):
        for j in range(NL):
            zbuf[i, pl.ds(j * 16, 16)] = zero16
        return 0
    lax.fori_loop(0, ZROWS, zrow, 0)
    for k in range(ROWS_PER_TILE_OUT // ZROWS):
        pltpu.sync_copy(
            zbuf, acc.at[pl.ds(s * ROWS_PER_TILE_OUT + k * ZROWS, ZROWS)])
    def xsrow(i, _):
        for j in range(NL):
            for b in range(NBUF):
                xs[b][i, pl.ds(j * 16, 16)] = zero16
        return 0
    lax.fori_loop(0, 16, xsrow, 0)

    pltpu.make_async_copy(batch_hbm.at[wid], ibuf, sx[NBUF - 1]).wait()

    def i4row(ch, _):
        ib4[ch, pl.ds(0, 16)] = ibuf[ch, pl.ds(0, 16)]
        return 0
    lax.fori_loop(0, NCHUNK, i4row, 0)
    plsc.subcore_barrier()

    def start_load(ch, b):
        pltpu.async_copy(x_hbm.at[wid, ch], xb[b], sx[b])

    def wait_load(ch, b):
        pltpu.make_async_copy(x_hbm.at[wid, ch], xb[b], sx[b]).wait()

    def start_scatter(ch, b):
        pltpu.async_copy(xb[b], acc.at[ibuf.at[ch]], ss[b], add=True)

    def wait_scatter(ch, b):
        pltpu.make_async_copy(xb[b], acc.at[ibuf.at[ch]], ss[b]).wait()

    def start_scatter_fast(ch, b):
        pltpu.async_copy(xs[b], acc.at[ib4.at[ch]], ss[b], add=True)

    def wait_scatter_fast(ch, b):
        pltpu.make_async_copy(xs[b], acc.at[ib4.at[ch]], ss[b]).wait()

    def single_id(ch):
        first = ibuf[ch, pl.ds(0, 16)][0]
        last = ibuf[ch, pl.ds(CHUNK - 16, 16)][15]
        return first == last

    def fast_accum(ch, b):
        init = (zero16,) * (NPART * NL)

        def rbody(r4, p):
            out = []
            for k in range(NPART):
                row = r4 * NPART + k
                for j in range(NL):
                    out.append(p[k * NL + j] + xb[b][row, pl.ds(j * 16, 16)])
            return tuple(out)

        p = lax.fori_loop(0, CHUNK // NPART, rbody, init)
        for k in range(NPART):
            for j in range(NL):
                xs[b][k, pl.ds(j * 16, 16)] = p[k * NL + j]
        start_scatter_fast(ch, b)

    def dispatch_scatter(ch, b):
        cond = single_id(ch)

        @pl.when(cond)
        def _():
            fast_accum(ch, b)

        @pl.when(jnp.logical_not(cond))
        def _():
            start_scatter(ch, b)

    def wait_scatter_any(ch, b):
        cond = single_id(ch)

        @pl.when(cond)
        def _():
            wait_scatter_fast(ch, b)

        @pl.when(jnp.logical_not(cond))
        def _():
            wait_scatter(ch, b)

    for p in range(NBUF - 1):
        start_load(p, p)

    def outer(k, _):
        for b in range(NBUF):
            ch = NBUF * k + b
            wait_load(ch, b)
            dispatch_scatter(ch, b)

            @pl.when(ch >= 1)
            def _():
                wait_scatter_any(ch - 1, (b - 1) % NBUF)

            @pl.when(ch + NBUF - 1 < NCHUNK)
            def _():
                start_load(ch + NBUF - 1, (b - 1) % NBUF)
        return 0

    lax.fori_loop(0, NCHUNK // NBUF, outer, 0)
    wait_scatter_any(NCHUNK - 1, (NCHUNK - 1) % NBUF)

    plsc.subcore_barrier()
    pltpu.sync_copy(
        acc.at[pl.ds(s * ROWS_PER_TILE_OUT, ROWS_PER_TILE_OUT)],
        out_hbm.at[c, pl.ds(s * ROWS_PER_TILE_OUT, ROWS_PER_TILE_OUT)],
    )


_sc_stage = functools.partial(
    pl.kernel,
    out_type=jax.ShapeDtypeStruct((NC, S, D), jnp.float32),
    mesh=plsc.VectorSubcoreMesh(core_axis_name="c", subcore_axis_name="s"),
    scratch_types=[
        pltpu.VMEM((CHUNK, D), jnp.float32),
        pltpu.VMEM((CHUNK, D), jnp.float32),
        pltpu.VMEM((CHUNK, D), jnp.float32),
        pltpu.VMEM((CHUNK, D), jnp.float32),
        pltpu.VMEM((CHUNK, D), jnp.float32),
        pltpu.VMEM((16, D), jnp.float32),
        pltpu.VMEM((16, D), jnp.float32),
        pltpu.VMEM((16, D), jnp.float32),
        pltpu.VMEM((16, D), jnp.float32),
        pltpu.VMEM((16, D), jnp.float32),
        pltpu.VMEM((NCHUNK, CHUNK), jnp.int32),
        pltpu.VMEM((NCHUNK, 16), jnp.int32),
        pltpu.VMEM((ZROWS, D), jnp.float32),
        pltpu.VMEM_SHARED((S, D), jnp.float32),
    ] + [pltpu.SemaphoreType.DMA] * 10,
)(_sc_body)


def _add_body(p_ref, o_ref):
    o_ref[...] = p_ref[0] + p_ref[1]


def kernel(x, batch):
    xr = x.reshape(NW, NCHUNK, CHUNK, D)
    br = batch.reshape(NW, NCHUNK, CHUNK)
    partials = _sc_stage(xr, br)
    out = pl.pallas_call(
        _add_body,
        out_shape=jax.ShapeDtypeStruct((S, D), jnp.float32),
    )(partials)
    return out
